# Initial kernel scaffold; baseline (speedup 1.0000x reference)
import jax
import jax.numpy as jnp
from jax.experimental import pallas as pl


def kernel(edge_index, edge_type, query_relation, query_entity, relation_embeddings, W1, b1, W2, b2, Wa1, ba1, Wa2, ba2, Wc1, bc1, Wc2, bc2):
    raise NotImplementedError("write your pallas kernel here")


# speedup vs baseline: 72.0104x; 72.0104x over previous
import functools

import jax
import jax.numpy as jnp
from jax import lax
from jax.experimental import pallas as pl
from jax.experimental.pallas import tpu as pltpu
from jax.experimental.pallas import tpu_sc as plsc

N_NODES = 50000
E = 1600000
K = 64
NC = 2
NS = 16
NW = NC * NS
LANE = 16

ROWS = 12512
E_PAD = ROWS * 128
BLK = 2048
NBLK = E_PAD // BLK
LAST_REAL_BLK = 781
MAX_BI = 25
N_PAD = 51200
SLICE_N = N_PAD // NS
CORE0_ROWS = 398 * 16


def _mesh():
    return plsc.VectorSubcoreMesh(core_axis_name="c", subcore_axis_name="s",
                                  num_cores=NC, num_subcores=NS)


def _worker(cid, sid):
    return cid * NS + sid


def _sched(w):
    start = 24 * w + jnp.minimum(w, 14)
    nblk = jnp.where(w < 14, 25, 24).astype(jnp.int32)
    return start, nblk


def _zero_ref(ref, nwords):
    def body(i, _):
        ref[pl.ds(i * LANE, LANE)] = jnp.zeros((LANE,), ref.dtype)
        return 0
    lax.fori_loop(0, nwords // LANE, body, 0)


def _k1_body(src2d, dst2d, degp, deg_sh, sidx, didx, ones_v, zero_v):
    cid = lax.axis_index("c")
    sid = lax.axis_index("s")
    w = _worker(cid, sid)
    _zero_ref(zero_v, SLICE_N)
    pltpu.sync_copy(zero_v, deg_sh.at[pl.ds(sid * SLICE_N, SLICE_N)])
    for i in range(8):
        ones_v[pl.ds(i * LANE, LANE)] = jnp.ones((LANE,), jnp.int32)
    plsc.subcore_barrier()

    start, nblk = _sched(w)

    def blk(bi, _):
        b = start + bi

        @pl.when(bi < nblk)
        def _():
            pltpu.sync_copy(src2d.at[pl.ds(b * 16, 16), :], sidx)
            pltpu.sync_copy(dst2d.at[pl.ds(b * 16, 16), :], didx)
            for r in range(16):
                if r < 4:
                    pltpu.sync_copy(ones_v, deg_sh.at[sidx.at[r]], add=True)
                    pltpu.sync_copy(ones_v, deg_sh.at[didx.at[r]], add=True)
                else:
                    @pl.when(b < LAST_REAL_BLK)
                    def _():
                        pltpu.sync_copy(ones_v, deg_sh.at[sidx.at[r]], add=True)
                        pltpu.sync_copy(ones_v, deg_sh.at[didx.at[r]], add=True)
        return 0

    lax.fori_loop(0, MAX_BI, blk, 0)
    plsc.subcore_barrier()
    pltpu.sync_copy(deg_sh.at[pl.ds(sid * SLICE_N, SLICE_N)],
                    degp.at[cid, pl.ds(sid * SLICE_N, SLICE_N)])


def _k1(src2d, dst2d):
    kern = pl.kernel(
        _k1_body,
        out_type=jax.ShapeDtypeStruct((NC, N_PAD), jnp.int32),
        mesh=_mesh(),
        compiler_params=pltpu.CompilerParams(needs_layout_passes=False),
        scratch_types=[
            pltpu.VMEM_SHARED((N_PAD,), jnp.int32),
            pltpu.VMEM((16, 128), jnp.int32),
            pltpu.VMEM((16, 128), jnp.int32),
            pltpu.VMEM((128,), jnp.int32),
            pltpu.VMEM((SLICE_N,), jnp.int32),
        ],
    )
    return kern(src2d, dst2d)


def _k2_body(degp, src1d, dst1d, iq1d, u1d, deg_v, a_v, b_v,
             sidx, didx, iqs, u_v):
    cid = lax.axis_index("c")
    sid = lax.axis_index("s")
    w = _worker(cid, sid)

    def mslice(j, _):
        off = j * SLICE_N
        pltpu.sync_copy(degp.at[0, pl.ds(off, SLICE_N)], a_v)
        pltpu.sync_copy(degp.at[1, pl.ds(off, SLICE_N)], b_v)

        def madd(i, _):
            s = pl.ds(i * LANE, LANE)
            deg_v[pl.ds(off + i * LANE, LANE)] = a_v[s] + b_v[s]
            return 0
        lax.fori_loop(0, SLICE_N // LANE, madd, 0)
        return 0
    lax.fori_loop(0, NS, mslice, 0)

    start, nblk = _sched(w)

    def blk(bi, _):
        b = start + bi

        @pl.when(bi < nblk)
        def _():
            eb = b * BLK
            pltpu.sync_copy(src1d.at[pl.ds(eb, BLK)], sidx)
            pltpu.sync_copy(dst1d.at[pl.ds(eb, BLK)], didx)
            pltpu.sync_copy(iq1d.at[pl.ds(eb, BLK)], iqs)

            def vec(i, _):
                s = pl.ds(i * LANE, LANE)
                ds16 = plsc.load_gather(deg_v, [sidx[s]])
                dd16 = plsc.load_gather(deg_v, [didx[s]])
                u_v[s] = jnp.where(iqs[s] != 0, ds16 + dd16 + 1, 0)
                return 0
            lax.fori_loop(0, BLK // LANE, vec, 0)
            pltpu.sync_copy(u_v, u1d.at[pl.ds(eb, BLK)])
        return 0

    lax.fori_loop(0, MAX_BI, blk, 0)


def _k2(degp, src1d, dst1d, iq1d):
    kern = pl.kernel(
        _k2_body,
        out_type=jax.ShapeDtypeStruct((E_PAD,), jnp.int32),
        mesh=_mesh(),
        compiler_params=pltpu.CompilerParams(needs_layout_passes=False),
        scratch_types=[
            pltpu.VMEM((N_PAD,), jnp.int32),
            pltpu.VMEM((SLICE_N,), jnp.int32),
            pltpu.VMEM((SLICE_N,), jnp.int32),
            pltpu.VMEM((BLK,), jnp.int32),
            pltpu.VMEM((BLK,), jnp.int32),
            pltpu.VMEM((BLK,), jnp.int32),
            pltpu.VMEM((BLK,), jnp.int32),
        ],
    )
    return kern(degp, src1d, dst1d, iq1d)


def _kc_body(u_ref, rel_ref, w1_ref, b1_ref, w2_ref, b2_ref, wa1_ref, ba1_ref,
             wa2_ref, ba2_ref, wc1_ref, bc1_ref, wc2_ref, bc2_ref,
             ti_ref, cv_ref):
    u = u_ref[...]

    def bs(_, lohi):
        lo, hi = lohi
        mid = (lo + hi) // 2
        cnt = jnp.sum((u > mid).astype(jnp.int32))
        take = cnt < K
        return (jnp.where(take, lo, mid + 1), jnp.where(take, mid, hi))

    t_val, _ = lax.fori_loop(0, 23, bs, (jnp.int32(0), jnp.int32(2 ** 23)))
    c_gt = jnp.sum((u > t_val).astype(jnp.int32))
    l_eff = jnp.where(t_val > 0, K - c_gt, 0)
    c0 = jnp.where(t_val > 0,
                   jnp.sum((u[:CORE0_ROWS, :] == t_val).astype(jnp.int32)),
                   0)
    ti_ref[0:1, :] = jnp.broadcast_to(t_val, (1, 128))
    ti_ref[1:2, :] = jnp.broadcast_to(l_eff, (1, 128))
    ti_ref[2:3, :] = jnp.broadcast_to(c0, (1, 128))
    ti_ref[3:8, :] = jnp.zeros((5, 128), jnp.int32)

    rel = rel_ref[...]
    h1 = jnp.maximum(jnp.dot(rel, w1_ref[...],
                             preferred_element_type=jnp.float32)
                     + b1_ref[...], 0.0)
    ctx = jnp.dot(h1, w2_ref[...],
                  preferred_element_type=jnp.float32) + b2_ref[...]
    cat = jnp.concatenate([ctx, rel], axis=1)
    a = jnp.maximum(jnp.dot(cat, wa1_ref[...],
                            preferred_element_type=jnp.float32)
                    + ba1_ref[...], 0.0)
    wgt = 1.0 / (1.0 + jnp.exp(-(jnp.dot(a, wa2_ref[...],
                                         preferred_element_type=jnp.float32)
                                 + ba2_ref[...])))
    fin = jnp.concatenate([ctx * wgt[0, 0], rel], axis=1)
    f1 = jnp.maximum(jnp.dot(fin, wc1_ref[...],
                             preferred_element_type=jnp.float32)
                     + bc1_ref[...], 0.0)
    fused = jnp.dot(f1, wc2_ref[...],
                    preferred_element_type=jnp.float32) + bc2_ref[...]
    c_out = 1.0 + jnp.tanh(jnp.mean(fused))
    cv_ref[...] = jnp.broadcast_to(c_out, (8, 128))


def _kc(u2d, rel_row, W1, b1, W2, b2, Wa1, ba1, Wa2, ba2, Wc1, bc1, Wc2, bc2):
    return pl.pallas_call(
        _kc_body,
        out_shape=[jax.ShapeDtypeStruct((8, 128), jnp.int32),
                   jax.ShapeDtypeStruct((8, 128), jnp.float32)],
    )(u2d, rel_row, W1, b1[None, :], W2, b2[None, :], Wa1, ba1[None, :],
      Wa2, ba2[None, :], Wc1, bc1[None, :], Wc2, bc2[None, :])


def _k4_body(src1d, dst1d, u1d, ti48, qe16, nmp, nm_sh, cnt_sh,
             sidx, didx, u_v, ti_v, qe_v, sc1, sc2, sc3, cb_v, cm_v, zero_v):
    cid = lax.axis_index("c")
    sid = lax.axis_index("s")
    w = _worker(cid, sid)
    _zero_ref(zero_v, SLICE_N)
    pltpu.sync_copy(zero_v, nm_sh.at[pl.ds(sid * SLICE_N, SLICE_N)])
    pltpu.sync_copy(ti48, ti_v)
    pltpu.sync_copy(qe16, qe_v)
    t_b = ti_v[pl.ds(0, LANE)]
    l_b = ti_v[pl.ds(16, LANE)]
    qe_b = qe_v[...]

    start, nblk = _sched(w)

    def blka(bi, tot):
        b = jnp.minimum(start + bi, NBLK - 1)
        eb = b * BLK
        pltpu.sync_copy(u1d.at[pl.ds(eb, BLK)], u_v)

        def vec(i, t16):
            s = pl.ds(i * LANE, LANE)
            return t16 + (u_v[s] == t_b).astype(jnp.int32)
        t16 = lax.fori_loop(0, BLK // LANE, vec, jnp.zeros((LANE,), jnp.int32))
        cnt = jnp.sum(t16)
        gate = (bi < nblk).astype(jnp.int32)
        return tot + cnt * gate

    my_eq = lax.fori_loop(0, MAX_BI, blka, jnp.int32(0))
    cb_v[...] = jnp.broadcast_to(my_eq, (LANE,))
    pltpu.sync_copy(cb_v, cnt_sh.at[sid])
    plsc.subcore_barrier()
    pltpu.sync_copy(cnt_sh, cm_v)
    c0row = ti_v[pl.ds(32, LANE)]
    base = jnp.where(cid > 0, c0row[0], 0)
    for t in range(NS):
        rowt = cm_v[t, :]
        base = base + jnp.where(t < sid, rowt[0], 0)

    def blkb(bi, eq_run):
        b = jnp.minimum(start + bi, NBLK - 1)
        gate = bi < nblk
        gate_i = gate.astype(jnp.int32)
        eb = b * BLK
        pltpu.sync_copy(src1d.at[pl.ds(eb, BLK)], sidx)
        pltpu.sync_copy(dst1d.at[pl.ds(eb, BLK)], didx)
        pltpu.sync_copy(u1d.at[pl.ds(eb, BLK)], u_v)

        def vec(i, run):
            s = pl.ds(i * LANE, LANE)
            sv = sidx[s]
            dv = didx[s]
            uu = u_v[s]
            eqm = (uu == t_b).astype(jnp.int32) * gate_i
            inc = plsc.cumsum(eqm)
            rank = jnp.broadcast_to(run, (LANE,)) + inc - 1
            contrib = ((uu > t_b) & jnp.broadcast_to(gate, (LANE,))) | \
                      ((eqm != 0) & (rank < l_b))
            v1 = ((sv == qe_b) & jnp.broadcast_to(gate, (LANE,))).astype(jnp.int32)
            v2 = ((dv == qe_b) & jnp.broadcast_to(gate, (LANE,))).astype(jnp.int32)
            vc = contrib.astype(jnp.int32)
            n1 = jnp.sum(v1)
            n2 = jnp.sum(v2)
            n3 = jnp.sum(vc)

            @pl.when(n1 > 0)
            def _():
                sc1[...] = v1
                pltpu.sync_copy(sc1, nm_sh.at[dv], add=True)

            @pl.when(n2 > 0)
            def _():
                sc2[...] = v2
                pltpu.sync_copy(sc2, nm_sh.at[sv], add=True)

            @pl.when(n3 > 0)
            def _():
                sc3[...] = vc
                pltpu.sync_copy(sc3, nm_sh.at[sv], add=True)
                pltpu.sync_copy(sc3, nm_sh.at[dv], add=True)

            return run + jnp.sum(eqm)

        return lax.fori_loop(0, BLK // LANE, vec, eq_run)

    lax.fori_loop(0, MAX_BI, blkb, base)
    plsc.subcore_barrier()
    pltpu.sync_copy(nm_sh.at[pl.ds(sid * SLICE_N, SLICE_N)],
                    nmp.at[cid, pl.ds(sid * SLICE_N, SLICE_N)])


def _k4(src1d, dst1d, u1d, ti48, qe16):
    kern = pl.kernel(
        _k4_body,
        out_type=jax.ShapeDtypeStruct((NC, N_PAD), jnp.int32),
        mesh=_mesh(),
        compiler_params=pltpu.CompilerParams(needs_layout_passes=False),
        scratch_types=[
            pltpu.VMEM_SHARED((N_PAD,), jnp.int32),
            pltpu.VMEM_SHARED((NS, 16), jnp.int32),
            pltpu.VMEM((BLK,), jnp.int32),
            pltpu.VMEM((BLK,), jnp.int32),
            pltpu.VMEM((BLK,), jnp.int32),
            pltpu.VMEM((48,), jnp.int32),
            pltpu.VMEM((16,), jnp.int32),
            pltpu.VMEM((LANE,), jnp.int32),
            pltpu.VMEM((LANE,), jnp.int32),
            pltpu.VMEM((LANE,), jnp.int32),
            pltpu.VMEM((LANE,), jnp.int32),
            pltpu.VMEM((NS, 16), jnp.int32),
            pltpu.VMEM((SLICE_N,), jnp.int32),
        ],
    )
    return kern(src1d, dst1d, u1d, ti48, qe16)


def _k5_body(nmp, src1d, dst1d, qe16, cv16, out1d, nm_v, a_v, b_v,
             sidx, didx, o_v, qe_v, cv_v):
    cid = lax.axis_index("c")
    sid = lax.axis_index("s")
    w = _worker(cid, sid)

    def mslice(j, _):
        off = j * SLICE_N
        pltpu.sync_copy(nmp.at[0, pl.ds(off, SLICE_N)], a_v)
        pltpu.sync_copy(nmp.at[1, pl.ds(off, SLICE_N)], b_v)

        def madd(i, _):
            s = pl.ds(i * LANE, LANE)
            nm_v[pl.ds(off + i * LANE, LANE)] = a_v[s] + b_v[s]
            return 0
        lax.fori_loop(0, SLICE_N // LANE, madd, 0)
        return 0
    lax.fori_loop(0, NS, mslice, 0)

    pltpu.sync_copy(qe16, qe_v)
    pltpu.sync_copy(cv16, cv_v)
    qe_b = qe_v[...]
    c_b = cv_v[...]
    zero16 = jnp.zeros((LANE,), jnp.float32)

    start, nblk = _sched(w)

    def blk(bi, _):
        b = start + bi

        @pl.when(bi < nblk)
        def _():
            eb = b * BLK
            pltpu.sync_copy(src1d.at[pl.ds(eb, BLK)], sidx)
            pltpu.sync_copy(dst1d.at[pl.ds(eb, BLK)], didx)

            def vec(i, _):
                s = pl.ds(i * LANE, LANE)
                sv = sidx[s]
                dv = didx[s]
                ns16 = plsc.load_gather(nm_v, [sv])
                nd16 = plsc.load_gather(nm_v, [dv])
                m = ((ns16 > 0) | (sv == qe_b)) & ((nd16 > 0) | (dv == qe_b))
                o_v[s] = jnp.where(m, c_b, zero16)
                return 0
            lax.fori_loop(0, BLK // LANE, vec, 0)
            pltpu.sync_copy(o_v, out1d.at[pl.ds(eb, BLK)])
        return 0

    lax.fori_loop(0, MAX_BI, blk, 0)


def _k5(nmp, src1d, dst1d, qe16, cv16):
    kern = pl.kernel(
        _k5_body,
        out_type=jax.ShapeDtypeStruct((E_PAD,), jnp.float32),
        mesh=_mesh(),
        compiler_params=pltpu.CompilerParams(needs_layout_passes=False),
        scratch_types=[
            pltpu.VMEM((N_PAD,), jnp.int32),
            pltpu.VMEM((SLICE_N,), jnp.int32),
            pltpu.VMEM((SLICE_N,), jnp.int32),
            pltpu.VMEM((BLK,), jnp.int32),
            pltpu.VMEM((BLK,), jnp.int32),
            pltpu.VMEM((BLK,), jnp.float32),
            pltpu.VMEM((16,), jnp.int32),
            pltpu.VMEM((16,), jnp.float32),
        ],
    )
    return kern(nmp, src1d, dst1d, qe16, cv16)


def kernel(edge_index, edge_type, query_relation, query_entity,
           relation_embeddings, W1, b1, W2, b2, Wa1, ba1, Wa2, ba2,
           Wc1, bc1, Wc2, bc2):
    src = edge_index[0].astype(jnp.int32)
    dst = edge_index[1].astype(jnp.int32)
    pad = E_PAD - E
    src_p = jnp.concatenate([src, jnp.zeros((pad,), jnp.int32)])
    dst_p = jnp.concatenate([dst, jnp.zeros((pad,), jnp.int32)])
    et_p = jnp.concatenate([edge_type.astype(jnp.int32),
                            jnp.full((pad,), -1, jnp.int32)])
    qr = jnp.asarray(query_relation, jnp.int32)
    qe = jnp.asarray(query_entity, jnp.int32)
    iq1d = (et_p == qr).astype(jnp.int32)
    src2d = src_p.reshape(ROWS, 128)
    dst2d = dst_p.reshape(ROWS, 128)
    qe16 = jnp.full((16,), qe, jnp.int32)

    degp = _k1(src2d, dst2d)
    u1d = _k2(degp, src_p, dst_p, iq1d)

    rel_row = jnp.take(relation_embeddings, qr, axis=0)[None, :]
    ti, cv = _kc(u1d.reshape(ROWS, 128), rel_row, W1, b1, W2, b2,
                 Wa1, ba1, Wa2, ba2, Wc1, bc1, Wc2, bc2)
    ti48 = ti[:3, :16].reshape(48)
    cv16 = cv[0, :16]

    nmp = _k4(src_p, dst_p, u1d, ti48, qe16)
    out_p = _k5(nmp, src_p, dst_p, qe16, cv16)
    return out_p[:E]


# speedup vs baseline: 128.5063x; 1.7846x over previous
import jax
import jax.numpy as jnp
from jax import lax
from jax.experimental import pallas as pl
from jax.experimental.pallas import tpu as pltpu
from jax.experimental.pallas import tpu_sc as plsc

N_NODES = 50000
E = 1600000
K = 64
NC = 2
NS = 16
NW = NC * NS
LANE = 16

ROWS = 12512
E_PAD = ROWS * 128
ROWS_ALLOC = 12560
E_ALLOC = ROWS_ALLOC * 128
BLK = 2048
NBLK = E_PAD // BLK
LAST_REAL_BLK = 781
MAX_BI = 25
GRP = 5
NGRP = MAX_BI // GRP
GBLK = GRP * BLK
N_PAD = 51200
SLICE_N = N_PAD // NS
HALF_N = N_PAD // 2


def _mesh():
    return plsc.VectorSubcoreMesh(core_axis_name="c", subcore_axis_name="s",
                                  num_cores=NC, num_subcores=NS)


_SC_PARAMS = dict(
    compiler_params=pltpu.CompilerParams(needs_layout_passes=False))


def _worker(cid, sid):
    return cid * NS + sid


def _sched(w):
    start = 24 * w + jnp.minimum(w, 14)
    nblk = jnp.where(w < 14, 25, 24).astype(jnp.int32)
    return start, nblk


def _sched_py(w):
    return 24 * w + min(w, 14), 24 + (1 if w < 14 else 0)


def _zero_ref(ref, nwords):
    def body(i, _):
        ref[pl.ds(i * LANE, LANE)] = jnp.zeros((LANE,), ref.dtype)
        return 0
    lax.fori_loop(0, nwords // LANE, body, 0)


def _k1_body(src2d, dst2d, degp, deg_sh, sidx, didx, ones_v, zero_v, sem):
    cid = lax.axis_index("c")
    sid = lax.axis_index("s")
    w = _worker(cid, sid)
    _zero_ref(zero_v, SLICE_N)
    pltpu.sync_copy(zero_v, deg_sh.at[pl.ds(sid * SLICE_N, SLICE_N)])
    for i in range(8):
        ones_v[pl.ds(i * LANE, LANE)] = jnp.ones((LANE,), jnp.int32)
    plsc.subcore_barrier()

    start, nblk = _sched(w)

    def grp(gi, _):
        gb = start + gi * GRP
        pltpu.sync_copy(src2d.at[pl.ds(gb * 16, 16 * GRP), :], sidx)
        pltpu.sync_copy(dst2d.at[pl.ds(gb * 16, 16 * GRP), :], didx)
        for bj in range(GRP):
            b = gb + bj
            bi = gi * GRP + bj

            @pl.when((bi < nblk) & (b < LAST_REAL_BLK))
            def _():
                descs = []
                for r in range(16):
                    rr = bj * 16 + r
                    descs.append(pltpu.async_copy(
                        ones_v, deg_sh.at[sidx.at[rr]], sem, add=True))
                    descs.append(pltpu.async_copy(
                        ones_v, deg_sh.at[didx.at[rr]], sem, add=True))
                for d in descs:
                    d.wait()

            @pl.when(b == LAST_REAL_BLK)
            def _():
                descs = []
                for r in range(4):
                    rr = bj * 16 + r
                    descs.append(pltpu.async_copy(
                        ones_v, deg_sh.at[sidx.at[rr]], sem, add=True))
                    descs.append(pltpu.async_copy(
                        ones_v, deg_sh.at[didx.at[rr]], sem, add=True))
                for d in descs:
                    d.wait()
        return 0

    lax.fori_loop(0, NGRP, grp, 0)
    plsc.subcore_barrier()
    pltpu.sync_copy(deg_sh.at[pl.ds(sid * SLICE_N, SLICE_N)],
                    degp.at[cid, pl.ds(sid * SLICE_N, SLICE_N)])


def _k1(src2d, dst2d):
    kern = pl.kernel(
        _k1_body,
        out_type=jax.ShapeDtypeStruct((NC, N_PAD), jnp.int32),
        mesh=_mesh(),
        scratch_types=[
            pltpu.VMEM_SHARED((N_PAD,), jnp.int32),
            pltpu.VMEM((16 * GRP, 128), jnp.int32),
            pltpu.VMEM((16 * GRP, 128), jnp.int32),
            pltpu.VMEM((128,), jnp.int32),
            pltpu.VMEM((SLICE_N,), jnp.int32),
            pltpu.SemaphoreType.DMA,
        ],
        **_SC_PARAMS,
    )
    return kern(src2d, dst2d)


def _merge_partials(parts, full_v, bh_v):
    pltpu.sync_copy(parts.at[0], full_v)
    for h in range(2):
        pltpu.sync_copy(parts.at[1, pl.ds(h * HALF_N, HALF_N)], bh_v)

        def madd(i, _):
            s = pl.ds(i * LANE, LANE)
            d = pl.ds(h * HALF_N + i * LANE, LANE)
            full_v[d] = full_v[d] + bh_v[s]
            return 0
        lax.fori_loop(0, HALF_N // LANE, madd, 0)


def _k2_body(degp, src1d, dst1d, iq1d, u1d, deg_v, bh_v, sidx, didx, iqs, u_v):
    cid = lax.axis_index("c")
    sid = lax.axis_index("s")
    w = _worker(cid, sid)
    _merge_partials(degp, deg_v, bh_v)

    start, nblk = _sched(w)

    def grp(gi, _):
        gb = start + gi * GRP
        es = gb * BLK
        pltpu.sync_copy(src1d.at[pl.ds(es, GBLK)], sidx)
        pltpu.sync_copy(dst1d.at[pl.ds(es, GBLK)], didx)
        pltpu.sync_copy(iq1d.at[pl.ds(es, GBLK)], iqs)

        def vec(i, _):
            s = pl.ds(i * LANE, LANE)
            ds16 = plsc.load_gather(deg_v, [sidx[s]])
            dd16 = plsc.load_gather(deg_v, [didx[s]])
            u_v[s] = jnp.where(iqs[s] != 0, ds16 + dd16 + 1, 0)
            return 0
        lax.fori_loop(0, GBLK // LANE, vec, 0)

        @pl.when(gi < NGRP - 1)
        def _():
            pltpu.sync_copy(u_v, u1d.at[pl.ds(es, GBLK)])

        @pl.when(gi == NGRP - 1)
        def _():
            pltpu.sync_copy(u_v.at[pl.ds(0, (GRP - 1) * BLK)],
                            u1d.at[pl.ds(es, (GRP - 1) * BLK)])

            @pl.when(GRP * (NGRP - 1) + GRP - 1 < nblk)
            def _():
                pltpu.sync_copy(
                    u_v.at[pl.ds((GRP - 1) * BLK, BLK)],
                    u1d.at[pl.ds(es + (GRP - 1) * BLK, BLK)])
        return 0

    lax.fori_loop(0, NGRP, grp, 0)


def _k2(degp, src1d, dst1d, iq1d):
    kern = pl.kernel(
        _k2_body,
        out_type=jax.ShapeDtypeStruct((E_ALLOC,), jnp.int32),
        mesh=_mesh(),
        scratch_types=[
            pltpu.VMEM((N_PAD,), jnp.int32),
            pltpu.VMEM((HALF_N,), jnp.int32),
            pltpu.VMEM((GBLK,), jnp.int32),
            pltpu.VMEM((GBLK,), jnp.int32),
            pltpu.VMEM((GBLK,), jnp.int32),
            pltpu.VMEM((GBLK,), jnp.int32),
        ],
        **_SC_PARAMS,
    )
    return kern(degp, src1d, dst1d, iq1d)


def _kc_body(u_ref, rel_ref, w1_ref, b1_ref, w2_ref, b2_ref, wa1_ref, ba1_ref,
             wa2_ref, ba2_ref, wc1_ref, bc1_ref, wc2_ref, bc2_ref,
             ti_ref, cv_ref):
    u = u_ref[...]
    m = jnp.max(u)

    def cond(lohi):
        return lohi[0] < lohi[1]

    def step(lohi):
        lo, hi = lohi
        mid = (lo + hi) // 2
        cnt = jnp.sum((u > mid).astype(jnp.int32))
        take = cnt < K
        return (jnp.where(take, lo, mid + 1), jnp.where(take, mid, hi))

    t_val, _ = lax.while_loop(cond, step, (jnp.int32(0), m))
    c_gt = jnp.sum((u > t_val).astype(jnp.int32))
    l_eff = jnp.where(t_val > 0, K - c_gt, 0)
    eq = jnp.logical_and(u == t_val, t_val > 0).astype(jnp.int32)
    iota128 = lax.broadcasted_iota(jnp.int32, (1, 128), 1)
    bases = jnp.zeros((1, 128), jnp.int32)
    run = jnp.int32(0)
    for wi in range(NW):
        st, nb = _sched_py(wi)
        bases = bases + jnp.where(iota128 == wi, run, 0)
        run = run + jnp.sum(eq[st * 16:(st + nb) * 16, :])
    ti_ref[0:1, :] = jnp.broadcast_to(t_val, (1, 128))
    ti_ref[1:2, :] = jnp.broadcast_to(l_eff, (1, 128))
    ti_ref[2:3, :] = bases
    ti_ref[3:8, :] = jnp.zeros((5, 128), jnp.int32)

    rel = rel_ref[...]
    h1 = jnp.maximum(jnp.dot(rel, w1_ref[...],
                             preferred_element_type=jnp.float32)
                     + b1_ref[...], 0.0)
    ctx = jnp.dot(h1, w2_ref[...],
                  preferred_element_type=jnp.float32) + b2_ref[...]
    cat = jnp.concatenate([ctx, rel], axis=1)
    a = jnp.maximum(jnp.dot(cat, wa1_ref[...],
                            preferred_element_type=jnp.float32)
                    + ba1_ref[...], 0.0)
    wgt = 1.0 / (1.0 + jnp.exp(-(jnp.dot(a, wa2_ref[...],
                                         preferred_element_type=jnp.float32)
                                 + ba2_ref[...])))
    fin = jnp.concatenate([ctx * wgt[0, 0], rel], axis=1)
    f1 = jnp.maximum(jnp.dot(fin, wc1_ref[...],
                             preferred_element_type=jnp.float32)
                     + bc1_ref[...], 0.0)
    fused = jnp.dot(f1, wc2_ref[...],
                    preferred_element_type=jnp.float32) + bc2_ref[...]
    c_out = 1.0 + jnp.tanh(jnp.mean(fused))
    cv_ref[...] = jnp.broadcast_to(c_out, (8, 128))


def _kc(u2d, rel_row, W1, b1, W2, b2, Wa1, ba1, Wa2, ba2, Wc1, bc1, Wc2, bc2):
    return pl.pallas_call(
        _kc_body,
        out_shape=[jax.ShapeDtypeStruct((8, 128), jnp.int32),
                   jax.ShapeDtypeStruct((8, 128), jnp.float32)],
    )(u2d, rel_row, W1, b1[None, :], W2, b2[None, :], Wa1, ba1[None, :],
      Wa2, ba2[None, :], Wc1, bc1[None, :], Wc2, bc2[None, :])


def _k4_body(src1d, dst1d, u1d, ti64, qe16, nmp, nm_sh,
             sidx, didx, u_v, ti_v, qe_v, sc1, sc2, sc3, run_v, zero_v):
    cid = lax.axis_index("c")
    sid = lax.axis_index("s")
    w = _worker(cid, sid)
    _zero_ref(zero_v, SLICE_N)
    pltpu.sync_copy(zero_v, nm_sh.at[pl.ds(sid * SLICE_N, SLICE_N)])
    pltpu.sync_copy(ti64, ti_v)
    pltpu.sync_copy(qe16, qe_v)
    t_b = ti_v[pl.ds(0, LANE)]
    l_b = ti_v[pl.ds(16, LANE)]
    tl_b = t_b + jnp.where(l_b > 0, 0, 1)
    brow = ti_v[pl.ds(32 + cid * LANE, LANE)]
    iv = lax.iota(jnp.int32, LANE)
    base = jnp.sum(jnp.where(iv == sid, brow, 0))
    run_v[...] = jnp.broadcast_to(base, (LANE,))
    qe_b = qe_v[...]
    plsc.subcore_barrier()

    start, nblk = _sched(w)

    def grp(gi, _):
        gb = start + gi * GRP
        es = gb * BLK
        pltpu.sync_copy(src1d.at[pl.ds(es, GBLK)], sidx)
        pltpu.sync_copy(dst1d.at[pl.ds(es, GBLK)], didx)
        pltpu.sync_copy(u1d.at[pl.ds(es, GBLK)], u_v)
        for bj in range(GRP):
            bi = gi * GRP + bj
            off = bj * BLK

            def sc(i, ah):
                s = pl.ds(off + i * LANE, LANE)
                sv = sidx[s]
                dv = didx[s]
                uu = u_v[s]
                h = (sv == qe_b) | (dv == qe_b) | (uu >= tl_b)
                return ah | h.astype(jnp.int32)

            ah = lax.fori_loop(0, BLK // LANE, sc,
                               jnp.zeros((LANE,), jnp.int32))
            nh = jnp.sum(ah)

            @pl.when((nh > 0) & (bi < nblk))
            def _():
                def vec(i, _):
                    s = pl.ds(off + i * LANE, LANE)
                    sv = sidx[s]
                    dv = didx[s]
                    uu = u_v[s]
                    eqm = (uu == t_b).astype(jnp.int32)
                    inc = plsc.cumsum(eqm)
                    run_b = run_v[...]
                    rank = run_b + inc - 1
                    contrib = (uu > t_b) | ((eqm != 0) & (rank < l_b))
                    v1 = (sv == qe_b).astype(jnp.int32)
                    v2 = (dv == qe_b).astype(jnp.int32)
                    vc = contrib.astype(jnp.int32)
                    n1 = jnp.sum(v1)
                    n2 = jnp.sum(v2)
                    n3 = jnp.sum(vc)

                    @pl.when(n1 > 0)
                    def _():
                        sc1[...] = v1
                        pltpu.sync_copy(sc1, nm_sh.at[dv], add=True)

                    @pl.when(n2 > 0)
                    def _():
                        sc2[...] = v2
                        pltpu.sync_copy(sc2, nm_sh.at[sv], add=True)

                    @pl.when(n3 > 0)
                    def _():
                        sc3[...] = vc
                        pltpu.sync_copy(sc3, nm_sh.at[sv], add=True)
                        pltpu.sync_copy(sc3, nm_sh.at[dv], add=True)

                    run_v[...] = run_b + jnp.broadcast_to(jnp.sum(eqm),
                                                          (LANE,))
                    return 0
                lax.fori_loop(0, BLK // LANE, vec, 0)
        return 0

    lax.fori_loop(0, NGRP, grp, 0)
    plsc.subcore_barrier()
    pltpu.sync_copy(nm_sh.at[pl.ds(sid * SLICE_N, SLICE_N)],
                    nmp.at[cid, pl.ds(sid * SLICE_N, SLICE_N)])


def _k4(src1d, dst1d, u1d, ti64, qe16):
    kern = pl.kernel(
        _k4_body,
        out_type=jax.ShapeDtypeStruct((NC, N_PAD), jnp.int32),
        mesh=_mesh(),
        scratch_types=[
            pltpu.VMEM_SHARED((N_PAD,), jnp.int32),
            pltpu.VMEM((GBLK,), jnp.int32),
            pltpu.VMEM((GBLK,), jnp.int32),
            pltpu.VMEM((GBLK,), jnp.int32),
            pltpu.VMEM((64,), jnp.int32),
            pltpu.VMEM((16,), jnp.int32),
            pltpu.VMEM((LANE,), jnp.int32),
            pltpu.VMEM((LANE,), jnp.int32),
            pltpu.VMEM((LANE,), jnp.int32),
            pltpu.VMEM((LANE,), jnp.int32),
            pltpu.VMEM((SLICE_N,), jnp.int32),
        ],
        **_SC_PARAMS,
    )
    return kern(src1d, dst1d, u1d, ti64, qe16)


def _k5_body(nmp, src1d, dst1d, qe16, cv16, out1d, nm_v, bh_v,
             sidx, didx, o_v, qe_v, cv_v):
    cid = lax.axis_index("c")
    sid = lax.axis_index("s")
    w = _worker(cid, sid)
    _merge_partials(nmp, nm_v, bh_v)
    pltpu.sync_copy(qe16, qe_v)
    pltpu.sync_copy(cv16, cv_v)
    qe_b = qe_v[...]
    c_b = cv_v[...]
    zero16 = jnp.zeros((LANE,), jnp.float32)

    start, nblk = _sched(w)

    def grp(gi, _):
        gb = start + gi * GRP
        es = gb * BLK
        pltpu.sync_copy(src1d.at[pl.ds(es, GBLK)], sidx)
        pltpu.sync_copy(dst1d.at[pl.ds(es, GBLK)], didx)

        def vec(i, _):
            s = pl.ds(i * LANE, LANE)
            sv = sidx[s]
            dv = didx[s]
            ns16 = plsc.load_gather(nm_v, [sv])
            nd16 = plsc.load_gather(nm_v, [dv])
            msk = ((ns16 > 0) | (sv == qe_b)) & ((nd16 > 0) | (dv == qe_b))
            o_v[s] = jnp.where(msk, c_b, zero16)
            return 0
        lax.fori_loop(0, GBLK // LANE, vec, 0)

        @pl.when(gi < NGRP - 1)
        def _():
            pltpu.sync_copy(o_v, out1d.at[pl.ds(es, GBLK)])

        @pl.when(gi == NGRP - 1)
        def _():
            pltpu.sync_copy(o_v.at[pl.ds(0, (GRP - 1) * BLK)],
                            out1d.at[pl.ds(es, (GRP - 1) * BLK)])

            @pl.when(GRP * (NGRP - 1) + GRP - 1 < nblk)
            def _():
                pltpu.sync_copy(
                    o_v.at[pl.ds((GRP - 1) * BLK, BLK)],
                    out1d.at[pl.ds(es + (GRP - 1) * BLK, BLK)])
        return 0

    lax.fori_loop(0, NGRP, grp, 0)


def _k5(nmp, src1d, dst1d, qe16, cv16):
    kern = pl.kernel(
        _k5_body,
        out_type=jax.ShapeDtypeStruct((E_ALLOC,), jnp.float32),
        mesh=_mesh(),
        scratch_types=[
            pltpu.VMEM((N_PAD,), jnp.int32),
            pltpu.VMEM((HALF_N,), jnp.int32),
            pltpu.VMEM((GBLK,), jnp.int32),
            pltpu.VMEM((GBLK,), jnp.int32),
            pltpu.VMEM((GBLK,), jnp.float32),
            pltpu.VMEM((16,), jnp.int32),
            pltpu.VMEM((16,), jnp.float32),
        ],
        **_SC_PARAMS,
    )
    return kern(nmp, src1d, dst1d, qe16, cv16)


def kernel(edge_index, edge_type, query_relation, query_entity,
           relation_embeddings, W1, b1, W2, b2, Wa1, ba1, Wa2, ba2,
           Wc1, bc1, Wc2, bc2):
    src = edge_index[0].astype(jnp.int32)
    dst = edge_index[1].astype(jnp.int32)
    pad = E_ALLOC - E
    src_p = jnp.concatenate([src, jnp.zeros((pad,), jnp.int32)])
    dst_p = jnp.concatenate([dst, jnp.zeros((pad,), jnp.int32)])
    et_p = jnp.concatenate([edge_type.astype(jnp.int32),
                            jnp.full((pad,), -1, jnp.int32)])
    qr = jnp.asarray(query_relation, jnp.int32)
    qe = jnp.asarray(query_entity, jnp.int32)
    iq1d = (et_p == qr).astype(jnp.int32)
    src2d = src_p.reshape(ROWS_ALLOC, 128)
    dst2d = dst_p.reshape(ROWS_ALLOC, 128)
    qe16 = jnp.full((16,), qe, jnp.int32)

    degp = _k1(src2d, dst2d)
    u1d = _k2(degp, src_p, dst_p, iq1d)

    rel_row = jnp.take(relation_embeddings, qr, axis=0)[None, :]
    ti, cv = _kc(u1d[:E_PAD].reshape(ROWS, 128), rel_row, W1, b1, W2, b2,
                 Wa1, ba1, Wa2, ba2, Wc1, bc1, Wc2, bc2)
    ti64 = jnp.concatenate([ti[0, :16], ti[1, :16], ti[2, :32]])
    cv16 = cv[0, :16]

    nmp = _k4(src_p, dst_p, u1d, ti64, qe16)
    out_p = _k5(nmp, src_p, dst_p, qe16, cv16)
    return out_p[:E]


# speedup vs baseline: 135.4956x; 1.0544x over previous
import jax
import jax.numpy as jnp
from jax import lax
from jax.experimental import pallas as pl
from jax.experimental.pallas import tpu as pltpu
from jax.experimental.pallas import tpu_sc as plsc

N_NODES = 50000
E = 1600000
K = 64
NC = 2
NS = 16
NW = NC * NS
LANE = 16

ROWS = 12512
E_PAD = ROWS * 128
ROWS_ALLOC = 12560
E_ALLOC = ROWS_ALLOC * 128
BLK = 2048
NBLK = E_PAD // BLK
LAST_REAL_BLK = 781
MAX_BI = 25
GRP = 5
NGRP = MAX_BI // GRP
GBLK = GRP * BLK
N_PAD = 51200
SLICE_N = N_PAD // NS
HALF_N = N_PAD // 2


def _mesh():
    return plsc.VectorSubcoreMesh(core_axis_name="c", subcore_axis_name="s",
                                  num_cores=NC, num_subcores=NS)


_SC_PARAMS = dict(
    compiler_params=pltpu.CompilerParams(needs_layout_passes=False))


def _worker(cid, sid):
    return cid * NS + sid


def _sched(w):
    start = 24 * w + jnp.minimum(w, 14)
    nblk = jnp.where(w < 14, 25, 24).astype(jnp.int32)
    return start, nblk


def _sched_py(w):
    return 24 * w + min(w, 14), 24 + (1 if w < 14 else 0)


def _zero_ref(ref, nwords):
    def body(i, _):
        ref[pl.ds(i * LANE, LANE)] = jnp.zeros((LANE,), ref.dtype)
        return 0
    lax.fori_loop(0, nwords // LANE, body, 0)


def _k1_body(src2d, dst2d, degp, deg_sh, sidx, didx, ones_v, zero_v, sem):
    cid = lax.axis_index("c")
    sid = lax.axis_index("s")
    w = _worker(cid, sid)
    _zero_ref(zero_v, SLICE_N)
    pltpu.sync_copy(zero_v, deg_sh.at[pl.ds(sid * SLICE_N, SLICE_N)])
    for i in range(8):
        ones_v[pl.ds(i * LANE, LANE)] = jnp.ones((LANE,), jnp.int32)
    plsc.subcore_barrier()

    start, nblk = _sched(w)

    def grp(gi, _):
        gb = start + gi * GRP
        pltpu.sync_copy(src2d.at[pl.ds(gb * 16, 16 * GRP), :], sidx)
        pltpu.sync_copy(dst2d.at[pl.ds(gb * 16, 16 * GRP), :], didx)
        for bj in range(GRP):
            b = gb + bj
            bi = gi * GRP + bj

            @pl.when((bi < nblk) & (b < LAST_REAL_BLK))
            def _():
                descs = []
                for r in range(16):
                    rr = bj * 16 + r
                    descs.append(pltpu.async_copy(
                        ones_v, deg_sh.at[sidx.at[rr]], sem, add=True))
                    descs.append(pltpu.async_copy(
                        ones_v, deg_sh.at[didx.at[rr]], sem, add=True))
                for d in descs:
                    d.wait()

            @pl.when(b == LAST_REAL_BLK)
            def _():
                descs = []
                for r in range(4):
                    rr = bj * 16 + r
                    descs.append(pltpu.async_copy(
                        ones_v, deg_sh.at[sidx.at[rr]], sem, add=True))
                    descs.append(pltpu.async_copy(
                        ones_v, deg_sh.at[didx.at[rr]], sem, add=True))
                for d in descs:
                    d.wait()
        return 0

    lax.fori_loop(0, NGRP, grp, 0)
    plsc.subcore_barrier()
    pltpu.sync_copy(deg_sh.at[pl.ds(sid * SLICE_N, SLICE_N)],
                    degp.at[cid, pl.ds(sid * SLICE_N, SLICE_N)])


def _k1(src2d, dst2d):
    kern = pl.kernel(
        _k1_body,
        out_type=jax.ShapeDtypeStruct((NC, N_PAD), jnp.int32),
        mesh=_mesh(),
        scratch_types=[
            pltpu.VMEM_SHARED((N_PAD,), jnp.int32),
            pltpu.VMEM((16 * GRP, 128), jnp.int32),
            pltpu.VMEM((16 * GRP, 128), jnp.int32),
            pltpu.VMEM((128,), jnp.int32),
            pltpu.VMEM((SLICE_N,), jnp.int32),
            pltpu.SemaphoreType.DMA,
        ],
        **_SC_PARAMS,
    )
    return kern(src2d, dst2d)


def _merge_partials(parts, full_v, bh_v):
    pltpu.sync_copy(parts.at[0], full_v)
    for h in range(2):
        pltpu.sync_copy(parts.at[1, pl.ds(h * HALF_N, HALF_N)], bh_v)

        def madd(i, _):
            s = pl.ds(i * LANE, LANE)
            d = pl.ds(h * HALF_N + i * LANE, LANE)
            full_v[d] = full_v[d] + bh_v[s]
            return 0
        lax.fori_loop(0, HALF_N // LANE, madd, 0)


IQ_BIT = 1 << 30


def _merge_chunked(parts, full_v, tmp_v, sem):
    pltpu.sync_copy(parts.at[0], full_v)
    for h in range(N_PAD // GBLK):
        pltpu.sync_copy(parts.at[1, pl.ds(h * GBLK, GBLK)], tmp_v)

        def madd(i, _):
            s = pl.ds(i * LANE, LANE)
            d = pl.ds(h * GBLK + i * LANE, LANE)
            full_v[d] = full_v[d] + tmp_v[s]
            return 0
        lax.fori_loop(0, GBLK // LANE, madd, 0)


def _k2_body(degp, senc1d, dst1d, u1d, deg_v, sA0, sA1, dA0, dA1,
             uo0, uo1, sem_i0, sem_i1, sem_o0, sem_o1):
    cid = lax.axis_index("c")
    sid = lax.axis_index("s")
    w = _worker(cid, sid)
    _merge_chunked(degp, deg_v, sA1, sem_i0)

    start, nblk = _sched(w)
    sbuf = [sA0, sA1]
    dbuf = [dA0, dA1]
    obuf = [uo0, uo1]

    sem_i = [sem_i0, sem_i1]
    sem_o = [sem_o0, sem_o1]

    def fire(gi):
        es = (start + gi * GRP) * BLK
        p = gi % 2
        return [
            pltpu.async_copy(senc1d.at[pl.ds(es, GBLK)], sbuf[p], sem_i[p]),
            pltpu.async_copy(dst1d.at[pl.ds(es, GBLK)], dbuf[p], sem_i[p])]

    pend_in = fire(0)
    pend_out = {0: [], 1: []}
    for gi in range(NGRP):
        p = gi % 2
        es = (start + gi * GRP) * BLK
        for d in pend_out[p]:
            d.wait()
        nxt = fire(gi + 1) if gi + 1 < NGRP else []
        for d in pend_in:
            d.wait()
        pend_in = nxt
        sA = sbuf[p]
        dA = dbuf[p]
        uo = obuf[p]

        def vec(i, _):
            s = pl.ds(i * LANE, LANE)
            se = sA[s]
            sv = se & (IQ_BIT - 1)
            ds16 = plsc.load_gather(deg_v, [sv])
            dd16 = plsc.load_gather(deg_v, [dA[s]])
            uo[s] = jnp.where(se >= IQ_BIT, ds16 + dd16 + 1, 0)
            return 0
        lax.fori_loop(0, GBLK // LANE, vec, 0)

        if gi < NGRP - 1:
            pend_out[p] = [pltpu.async_copy(uo, u1d.at[pl.ds(es, GBLK)],
                                            sem_o[p])]
        else:
            pend_out[p] = [pltpu.async_copy(
                uo.at[pl.ds(0, (GRP - 1) * BLK)],
                u1d.at[pl.ds(es, (GRP - 1) * BLK)], sem_o[p])]

            @pl.when(GRP * (NGRP - 1) + GRP - 1 < nblk)
            def _():
                pltpu.sync_copy(
                    uo.at[pl.ds((GRP - 1) * BLK, BLK)],
                    u1d.at[pl.ds(es + (GRP - 1) * BLK, BLK)])
    for p in (0, 1):
        for d in pend_out[p]:
            d.wait()


def _k2(degp, senc1d, dst1d):
    kern = pl.kernel(
        _k2_body,
        out_type=jax.ShapeDtypeStruct((E_ALLOC,), jnp.int32),
        mesh=_mesh(),
        scratch_types=[
            pltpu.VMEM((N_PAD,), jnp.int32),
            pltpu.VMEM((GBLK,), jnp.int32),
            pltpu.VMEM((GBLK,), jnp.int32),
            pltpu.VMEM((GBLK,), jnp.int32),
            pltpu.VMEM((GBLK,), jnp.int32),
            pltpu.VMEM((GBLK,), jnp.int32),
            pltpu.VMEM((GBLK,), jnp.int32),
            pltpu.SemaphoreType.DMA,
            pltpu.SemaphoreType.DMA,
            pltpu.SemaphoreType.DMA,
            pltpu.SemaphoreType.DMA,
        ],
        **_SC_PARAMS,
    )
    return kern(degp, senc1d, dst1d)


def _kc_body(u_ref, rel_ref, w1_ref, b1_ref, w2_ref, b2_ref, wa1_ref, ba1_ref,
             wa2_ref, ba2_ref, wc1_ref, bc1_ref, wc2_ref, bc2_ref,
             ti_ref, cv_ref):
    u = u_ref[...]
    m = jnp.max(u)

    def cond(lohi):
        return lohi[0] < lohi[1]

    def step(lohi):
        lo, hi = lohi
        mid = (lo + hi) // 2
        cnt = jnp.sum((u > mid).astype(jnp.int32))
        take = cnt < K
        return (jnp.where(take, lo, mid + 1), jnp.where(take, mid, hi))

    t_val, _ = lax.while_loop(cond, step, (jnp.int32(0), m))
    c_gt = jnp.sum((u > t_val).astype(jnp.int32))
    l_eff = jnp.where(t_val > 0, K - c_gt, 0)
    eq = jnp.logical_and(u == t_val, t_val > 0).astype(jnp.int32)
    iota128 = lax.broadcasted_iota(jnp.int32, (1, 128), 1)
    bases = jnp.zeros((1, 128), jnp.int32)
    run = jnp.int32(0)
    for wi in range(NW):
        st, nb = _sched_py(wi)
        bases = bases + jnp.where(iota128 == wi, run, 0)
        run = run + jnp.sum(eq[st * 16:(st + nb) * 16, :])
    ti_ref[0:1, :] = jnp.broadcast_to(t_val, (1, 128))
    ti_ref[1:2, :] = jnp.broadcast_to(l_eff, (1, 128))
    ti_ref[2:3, :] = bases
    ti_ref[3:8, :] = jnp.zeros((5, 128), jnp.int32)

    rel = rel_ref[...]
    h1 = jnp.maximum(jnp.dot(rel, w1_ref[...],
                             preferred_element_type=jnp.float32)
                     + b1_ref[...], 0.0)
    ctx = jnp.dot(h1, w2_ref[...],
                  preferred_element_type=jnp.float32) + b2_ref[...]
    cat = jnp.concatenate([ctx, rel], axis=1)
    a = jnp.maximum(jnp.dot(cat, wa1_ref[...],
                            preferred_element_type=jnp.float32)
                    + ba1_ref[...], 0.0)
    wgt = 1.0 / (1.0 + jnp.exp(-(jnp.dot(a, wa2_ref[...],
                                         preferred_element_type=jnp.float32)
                                 + ba2_ref[...])))
    fin = jnp.concatenate([ctx * wgt[0, 0], rel], axis=1)
    f1 = jnp.maximum(jnp.dot(fin, wc1_ref[...],
                             preferred_element_type=jnp.float32)
                     + bc1_ref[...], 0.0)
    fused = jnp.dot(f1, wc2_ref[...],
                    preferred_element_type=jnp.float32) + bc2_ref[...]
    c_out = 1.0 + jnp.tanh(jnp.mean(fused))
    cv_ref[...] = jnp.broadcast_to(c_out, (8, 128))


def _kc(u2d, rel_row, W1, b1, W2, b2, Wa1, ba1, Wa2, ba2, Wc1, bc1, Wc2, bc2):
    return pl.pallas_call(
        _kc_body,
        out_shape=[jax.ShapeDtypeStruct((8, 128), jnp.int32),
                   jax.ShapeDtypeStruct((8, 128), jnp.float32)],
    )(u2d, rel_row, W1, b1[None, :], W2, b2[None, :], Wa1, ba1[None, :],
      Wa2, ba2[None, :], Wc1, bc1[None, :], Wc2, bc2[None, :])


def _k4_body(src1d, dst1d, u1d, ti64, qe16, nmp, nm_sh,
             sidx, didx, u_v, ti_v, qe_v, sc1, sc2, sc3, run_v, zero_v):
    cid = lax.axis_index("c")
    sid = lax.axis_index("s")
    w = _worker(cid, sid)
    _zero_ref(zero_v, SLICE_N)
    pltpu.sync_copy(zero_v, nm_sh.at[pl.ds(sid * SLICE_N, SLICE_N)])
    pltpu.sync_copy(ti64, ti_v)
    pltpu.sync_copy(qe16, qe_v)
    t_b = ti_v[pl.ds(0, LANE)]
    l_b = ti_v[pl.ds(16, LANE)]
    tl_b = t_b + jnp.where(l_b > 0, 0, 1)
    brow = ti_v[pl.ds(32 + cid * LANE, LANE)]
    iv = lax.iota(jnp.int32, LANE)
    base = jnp.sum(jnp.where(iv == sid, brow, 0))
    run_v[...] = jnp.broadcast_to(base, (LANE,))
    qe_b = qe_v[...]
    plsc.subcore_barrier()

    start, nblk = _sched(w)

    def grp(gi, _):
        gb = start + gi * GRP
        es = gb * BLK
        pltpu.sync_copy(src1d.at[pl.ds(es, GBLK)], sidx)
        pltpu.sync_copy(dst1d.at[pl.ds(es, GBLK)], didx)
        pltpu.sync_copy(u1d.at[pl.ds(es, GBLK)], u_v)
        for bj in range(GRP):
            bi = gi * GRP + bj
            off = bj * BLK

            def sc(i, ah):
                s = pl.ds(off + i * LANE, LANE)
                sv = sidx[s]
                dv = didx[s]
                uu = u_v[s]
                h = (sv == qe_b) | (dv == qe_b) | (uu >= tl_b)
                return ah | h.astype(jnp.int32)

            ah = lax.fori_loop(0, BLK // LANE, sc,
                               jnp.zeros((LANE,), jnp.int32))
            nh = jnp.sum(ah)

            @pl.when((nh > 0) & (bi < nblk))
            def _():
                def vec(i, _):
                    s = pl.ds(off + i * LANE, LANE)
                    sv = sidx[s]
                    dv = didx[s]
                    uu = u_v[s]
                    eqm = (uu == t_b).astype(jnp.int32)
                    inc = plsc.cumsum(eqm)
                    run_b = run_v[...]
                    rank = run_b + inc - 1
                    contrib = (uu > t_b) | ((eqm != 0) & (rank < l_b))
                    v1 = (sv == qe_b).astype(jnp.int32)
                    v2 = (dv == qe_b).astype(jnp.int32)
                    vc = contrib.astype(jnp.int32)
                    n1 = jnp.sum(v1)
                    n2 = jnp.sum(v2)
                    n3 = jnp.sum(vc)

                    @pl.when(n1 > 0)
                    def _():
                        sc1[...] = v1
                        pltpu.sync_copy(sc1, nm_sh.at[dv], add=True)

                    @pl.when(n2 > 0)
                    def _():
                        sc2[...] = v2
                        pltpu.sync_copy(sc2, nm_sh.at[sv], add=True)

                    @pl.when(n3 > 0)
                    def _():
                        sc3[...] = vc
                        pltpu.sync_copy(sc3, nm_sh.at[sv], add=True)
                        pltpu.sync_copy(sc3, nm_sh.at[dv], add=True)

                    run_v[...] = run_b + jnp.broadcast_to(jnp.sum(eqm),
                                                          (LANE,))
                    return 0
                lax.fori_loop(0, BLK // LANE, vec, 0)
        return 0

    lax.fori_loop(0, NGRP, grp, 0)
    plsc.subcore_barrier()
    pltpu.sync_copy(nm_sh.at[pl.ds(sid * SLICE_N, SLICE_N)],
                    nmp.at[cid, pl.ds(sid * SLICE_N, SLICE_N)])


def _k4(src1d, dst1d, u1d, ti64, qe16):
    kern = pl.kernel(
        _k4_body,
        out_type=jax.ShapeDtypeStruct((NC, N_PAD), jnp.int32),
        mesh=_mesh(),
        scratch_types=[
            pltpu.VMEM_SHARED((N_PAD,), jnp.int32),
            pltpu.VMEM((GBLK,), jnp.int32),
            pltpu.VMEM((GBLK,), jnp.int32),
            pltpu.VMEM((GBLK,), jnp.int32),
            pltpu.VMEM((64,), jnp.int32),
            pltpu.VMEM((16,), jnp.int32),
            pltpu.VMEM((LANE,), jnp.int32),
            pltpu.VMEM((LANE,), jnp.int32),
            pltpu.VMEM((LANE,), jnp.int32),
            pltpu.VMEM((LANE,), jnp.int32),
            pltpu.VMEM((SLICE_N,), jnp.int32),
        ],
        **_SC_PARAMS,
    )
    return kern(src1d, dst1d, u1d, ti64, qe16)


def _k5_body(nmp, src1d, dst1d, qe16, cv16, out1d, nm_v, sA0, sA1, dA0, dA1,
             oo0, oo1, qe_v, cv_v, sem_i0, sem_i1, sem_o0, sem_o1):
    cid = lax.axis_index("c")
    sid = lax.axis_index("s")
    w = _worker(cid, sid)
    _merge_chunked(nmp, nm_v, sA1, sem_i0)
    pltpu.sync_copy(qe16, qe_v)
    pltpu.sync_copy(cv16, cv_v)
    qe_b = qe_v[...]
    c_b = cv_v[...]
    zero16 = jnp.zeros((LANE,), jnp.float32)

    start, nblk = _sched(w)
    sbuf = [sA0, sA1]
    dbuf = [dA0, dA1]
    obuf = [oo0, oo1]
    sem_i = [sem_i0, sem_i1]
    sem_o = [sem_o0, sem_o1]

    def fire(gi):
        es = (start + gi * GRP) * BLK
        p = gi % 2
        return [
            pltpu.async_copy(src1d.at[pl.ds(es, GBLK)], sbuf[p], sem_i[p]),
            pltpu.async_copy(dst1d.at[pl.ds(es, GBLK)], dbuf[p], sem_i[p])]

    pend_in = fire(0)
    pend_out = {0: [], 1: []}
    for gi in range(NGRP):
        p = gi % 2
        es = (start + gi * GRP) * BLK
        for d in pend_out[p]:
            d.wait()
        nxt = fire(gi + 1) if gi + 1 < NGRP else []
        for d in pend_in:
            d.wait()
        pend_in = nxt
        sA = sbuf[p]
        dA = dbuf[p]
        oo = obuf[p]

        def vec(i, _):
            s = pl.ds(i * LANE, LANE)
            sv = sA[s]
            dv = dA[s]
            ns16 = plsc.load_gather(nm_v, [sv])
            nd16 = plsc.load_gather(nm_v, [dv])
            msk = ((ns16 > 0) | (sv == qe_b)) & ((nd16 > 0) | (dv == qe_b))
            oo[s] = jnp.where(msk, c_b, zero16)
            return 0
        lax.fori_loop(0, GBLK // LANE, vec, 0)

        if gi < NGRP - 1:
            pend_out[p] = [pltpu.async_copy(oo, out1d.at[pl.ds(es, GBLK)],
                                            sem_o[p])]
        else:
            pend_out[p] = [pltpu.async_copy(
                oo.at[pl.ds(0, (GRP - 1) * BLK)],
                out1d.at[pl.ds(es, (GRP - 1) * BLK)], sem_o[p])]

            @pl.when(GRP * (NGRP - 1) + GRP - 1 < nblk)
            def _():
                pltpu.sync_copy(
                    oo.at[pl.ds((GRP - 1) * BLK, BLK)],
                    out1d.at[pl.ds(es + (GRP - 1) * BLK, BLK)])
    for p in (0, 1):
        for d in pend_out[p]:
            d.wait()


def _k5(nmp, src1d, dst1d, qe16, cv16):
    kern = pl.kernel(
        _k5_body,
        out_type=jax.ShapeDtypeStruct((E_ALLOC,), jnp.float32),
        mesh=_mesh(),
        scratch_types=[
            pltpu.VMEM((N_PAD,), jnp.int32),
            pltpu.VMEM((GBLK,), jnp.int32),
            pltpu.VMEM((GBLK,), jnp.int32),
            pltpu.VMEM((GBLK,), jnp.int32),
            pltpu.VMEM((GBLK,), jnp.int32),
            pltpu.VMEM((GBLK,), jnp.float32),
            pltpu.VMEM((GBLK,), jnp.float32),
            pltpu.VMEM((16,), jnp.int32),
            pltpu.VMEM((16,), jnp.float32),
            pltpu.SemaphoreType.DMA,
            pltpu.SemaphoreType.DMA,
            pltpu.SemaphoreType.DMA,
            pltpu.SemaphoreType.DMA,
        ],
        **_SC_PARAMS,
    )
    return kern(nmp, src1d, dst1d, qe16, cv16)


def kernel(edge_index, edge_type, query_relation, query_entity,
           relation_embeddings, W1, b1, W2, b2, Wa1, ba1, Wa2, ba2,
           Wc1, bc1, Wc2, bc2):
    src = edge_index[0].astype(jnp.int32)
    dst = edge_index[1].astype(jnp.int32)
    pad = E_ALLOC - E
    src_p = jnp.concatenate([src, jnp.zeros((pad,), jnp.int32)])
    dst_p = jnp.concatenate([dst, jnp.zeros((pad,), jnp.int32)])
    et_p = jnp.concatenate([edge_type.astype(jnp.int32),
                            jnp.full((pad,), -1, jnp.int32)])
    qr = jnp.asarray(query_relation, jnp.int32)
    qe = jnp.asarray(query_entity, jnp.int32)
    iq1d = (et_p == qr).astype(jnp.int32)
    senc = src_p | (iq1d * IQ_BIT)
    src2d = src_p.reshape(ROWS_ALLOC, 128)
    dst2d = dst_p.reshape(ROWS_ALLOC, 128)
    qe16 = jnp.full((16,), qe, jnp.int32)

    degp = _k1(src2d, dst2d)
    u1d = _k2(degp, senc, dst_p)

    rel_row = jnp.take(relation_embeddings, qr, axis=0)[None, :]
    ti, cv = _kc(u1d[:E_PAD].reshape(ROWS, 128), rel_row, W1, b1, W2, b2,
                 Wa1, ba1, Wa2, ba2, Wc1, bc1, Wc2, bc2)
    ti64 = jnp.concatenate([ti[0, :16], ti[1, :16], ti[2, :32]])
    cv16 = cv[0, :16]

    nmp = _k4(src_p, dst_p, u1d, ti64, qe16)
    out_p = _k5(nmp, src_p, dst_p, qe16, cv16)
    return out_p[:E]


# speedup vs baseline: 137.8797x; 1.0176x over previous
import jax
import jax.numpy as jnp
from jax import lax
from jax.experimental import pallas as pl
from jax.experimental.pallas import tpu as pltpu
from jax.experimental.pallas import tpu_sc as plsc

N_NODES = 50000
E = 1600000
K = 64
NC = 2
NS = 16
NW = NC * NS
LANE = 16

ROWS = 12512
E_PAD = ROWS * 128
ROWS_ALLOC = 12560
E_ALLOC = ROWS_ALLOC * 128
BLK = 2048
NBLK = E_PAD // BLK
LAST_REAL_BLK = 781
MAX_BI = 25
GRP = 5
NGRP = MAX_BI // GRP
GBLK = GRP * BLK
N_PAD = 51200
SLICE_N = N_PAD // NS
HALF_N = N_PAD // 2


def _mesh():
    return plsc.VectorSubcoreMesh(core_axis_name="c", subcore_axis_name="s",
                                  num_cores=NC, num_subcores=NS)


_SC_PARAMS = dict(
    compiler_params=pltpu.CompilerParams(needs_layout_passes=False))


def _worker(cid, sid):
    return cid * NS + sid


def _sched(w):
    start = 24 * w + jnp.minimum(w, 14)
    nblk = jnp.where(w < 14, 25, 24).astype(jnp.int32)
    return start, nblk


def _sched_py(w):
    return 24 * w + min(w, 14), 24 + (1 if w < 14 else 0)


def _zero_ref(ref, nwords):
    def body(i, _):
        ref[pl.ds(i * LANE, LANE)] = jnp.zeros((LANE,), ref.dtype)
        return 0
    lax.fori_loop(0, nwords // LANE, body, 0)


def _k1_body(src2d, dst2d, degp, deg_sh, sidx, didx, ones_v, zero_v, sem):
    cid = lax.axis_index("c")
    sid = lax.axis_index("s")
    w = _worker(cid, sid)
    _zero_ref(zero_v, SLICE_N)
    pltpu.sync_copy(zero_v, deg_sh.at[pl.ds(sid * SLICE_N, SLICE_N)])
    for i in range(8):
        ones_v[pl.ds(i * LANE, LANE)] = jnp.ones((LANE,), jnp.int32)
    plsc.subcore_barrier()

    start, nblk = _sched(w)

    def grp(gi, _):
        gb = start + gi * GRP
        pltpu.sync_copy(src2d.at[pl.ds(gb * 16, 16 * GRP), :], sidx)
        pltpu.sync_copy(dst2d.at[pl.ds(gb * 16, 16 * GRP), :], didx)
        for bj in range(GRP):
            b = gb + bj
            bi = gi * GRP + bj

            @pl.when((bi < nblk) & (b < LAST_REAL_BLK))
            def _():
                descs = []
                for r in range(16):
                    rr = bj * 16 + r
                    descs.append(pltpu.async_copy(
                        ones_v, deg_sh.at[sidx.at[rr]], sem, add=True))
                    descs.append(pltpu.async_copy(
                        ones_v, deg_sh.at[didx.at[rr]], sem, add=True))
                for d in descs:
                    d.wait()

            @pl.when(b == LAST_REAL_BLK)
            def _():
                descs = []
                for r in range(4):
                    rr = bj * 16 + r
                    descs.append(pltpu.async_copy(
                        ones_v, deg_sh.at[sidx.at[rr]], sem, add=True))
                    descs.append(pltpu.async_copy(
                        ones_v, deg_sh.at[didx.at[rr]], sem, add=True))
                for d in descs:
                    d.wait()
        return 0

    lax.fori_loop(0, NGRP, grp, 0)
    plsc.subcore_barrier()
    pltpu.sync_copy(deg_sh.at[pl.ds(sid * SLICE_N, SLICE_N)],
                    degp.at[cid, pl.ds(sid * SLICE_N, SLICE_N)])


def _k1(src2d, dst2d):
    kern = pl.kernel(
        _k1_body,
        out_type=jax.ShapeDtypeStruct((NC, N_PAD), jnp.int32),
        mesh=_mesh(),
        scratch_types=[
            pltpu.VMEM_SHARED((N_PAD,), jnp.int32),
            pltpu.VMEM((16 * GRP, 128), jnp.int32),
            pltpu.VMEM((16 * GRP, 128), jnp.int32),
            pltpu.VMEM((128,), jnp.int32),
            pltpu.VMEM((SLICE_N,), jnp.int32),
            pltpu.SemaphoreType.DMA,
        ],
        **_SC_PARAMS,
    )
    return kern(src2d, dst2d)


def _merge_partials(parts, full_v, bh_v):
    pltpu.sync_copy(parts.at[0], full_v)
    for h in range(2):
        pltpu.sync_copy(parts.at[1, pl.ds(h * HALF_N, HALF_N)], bh_v)

        def madd(i, _):
            s = pl.ds(i * LANE, LANE)
            d = pl.ds(h * HALF_N + i * LANE, LANE)
            full_v[d] = full_v[d] + bh_v[s]
            return 0
        lax.fori_loop(0, HALF_N // LANE, madd, 0)


IQ_BIT = 1 << 30


def _merge_chunked(parts, full_v, tmp_v, sem):
    pltpu.sync_copy(parts.at[0], full_v)
    for h in range(N_PAD // GBLK):
        pltpu.sync_copy(parts.at[1, pl.ds(h * GBLK, GBLK)], tmp_v)

        def madd(i, _):
            s = pl.ds(i * LANE, LANE)
            d = pl.ds(h * GBLK + i * LANE, LANE)
            full_v[d] = full_v[d] + tmp_v[s]
            return 0
        lax.fori_loop(0, GBLK // LANE, madd, 0)


def _k2_body(degp, senc1d, dst1d, u1d, deg_v, sA0, sA1, dA0, dA1,
             uo0, uo1, sem_i0, sem_i1, sem_o0, sem_o1):
    cid = lax.axis_index("c")
    sid = lax.axis_index("s")
    w = _worker(cid, sid)
    _merge_chunked(degp, deg_v, sA1, sem_i0)

    start, nblk = _sched(w)
    sbuf = [sA0, sA1]
    dbuf = [dA0, dA1]
    obuf = [uo0, uo1]

    sem_i = [sem_i0, sem_i1]
    sem_o = [sem_o0, sem_o1]

    def fire(gi):
        es = (start + gi * GRP) * BLK
        p = gi % 2
        return [
            pltpu.async_copy(senc1d.at[pl.ds(es, GBLK)], sbuf[p], sem_i[p]),
            pltpu.async_copy(dst1d.at[pl.ds(es, GBLK)], dbuf[p], sem_i[p])]

    pend_in = fire(0)
    pend_out = {0: [], 1: []}
    for gi in range(NGRP):
        p = gi % 2
        es = (start + gi * GRP) * BLK
        for d in pend_out[p]:
            d.wait()
        nxt = fire(gi + 1) if gi + 1 < NGRP else []
        for d in pend_in:
            d.wait()
        pend_in = nxt
        sA = sbuf[p]
        dA = dbuf[p]
        uo = obuf[p]

        def vec(i, _):
            s = pl.ds(i * LANE, LANE)
            se = sA[s]
            sv = se & (IQ_BIT - 1)
            ds16 = plsc.load_gather(deg_v, [sv])
            dd16 = plsc.load_gather(deg_v, [dA[s]])
            uo[s] = jnp.where(se >= IQ_BIT, ds16 + dd16 + 1, 0)
            return 0
        lax.fori_loop(0, GBLK // LANE, vec, 0)

        if gi < NGRP - 1:
            pend_out[p] = [pltpu.async_copy(uo, u1d.at[pl.ds(es, GBLK)],
                                            sem_o[p])]
        else:
            pend_out[p] = [pltpu.async_copy(
                uo.at[pl.ds(0, (GRP - 1) * BLK)],
                u1d.at[pl.ds(es, (GRP - 1) * BLK)], sem_o[p])]

            @pl.when(GRP * (NGRP - 1) + GRP - 1 < nblk)
            def _():
                pltpu.sync_copy(
                    uo.at[pl.ds((GRP - 1) * BLK, BLK)],
                    u1d.at[pl.ds(es + (GRP - 1) * BLK, BLK)])
    for p in (0, 1):
        for d in pend_out[p]:
            d.wait()


def _k2(degp, senc1d, dst1d):
    kern = pl.kernel(
        _k2_body,
        out_type=jax.ShapeDtypeStruct((E_ALLOC,), jnp.int32),
        mesh=_mesh(),
        scratch_types=[
            pltpu.VMEM((N_PAD,), jnp.int32),
            pltpu.VMEM((GBLK,), jnp.int32),
            pltpu.VMEM((GBLK,), jnp.int32),
            pltpu.VMEM((GBLK,), jnp.int32),
            pltpu.VMEM((GBLK,), jnp.int32),
            pltpu.VMEM((GBLK,), jnp.int32),
            pltpu.VMEM((GBLK,), jnp.int32),
            pltpu.SemaphoreType.DMA,
            pltpu.SemaphoreType.DMA,
            pltpu.SemaphoreType.DMA,
            pltpu.SemaphoreType.DMA,
        ],
        **_SC_PARAMS,
    )
    return kern(degp, senc1d, dst1d)


def _kc_body(u_ref, rel_ref, w1_ref, b1_ref, w2_ref, b2_ref, wa1_ref, ba1_ref,
             wa2_ref, ba2_ref, wc1_ref, bc1_ref, wc2_ref, bc2_ref,
             ti_ref, cv_ref):
    u = u_ref[...]
    m = jnp.max(u)

    def cond(lohi):
        return lohi[0] < lohi[1]

    def step(lohi):
        lo, hi = lohi
        mid = (lo + hi) // 2
        cnt = jnp.sum((u > mid).astype(jnp.int32))
        take = cnt < K
        return (jnp.where(take, lo, mid + 1), jnp.where(take, mid, hi))

    t_val, _ = lax.while_loop(cond, step, (jnp.int32(0), m))
    c_gt = jnp.sum((u > t_val).astype(jnp.int32))
    l_eff = jnp.where(t_val > 0, K - c_gt, 0)
    eq = jnp.logical_and(u == t_val, t_val > 0).astype(jnp.int32)
    iota128 = lax.broadcasted_iota(jnp.int32, (1, 128), 1)
    bases = jnp.zeros((1, 128), jnp.int32)
    run = jnp.int32(0)
    for wi in range(NW):
        st, nb = _sched_py(wi)
        bases = bases + jnp.where(iota128 == wi, run, 0)
        run = run + jnp.sum(eq[st * 16:(st + nb) * 16, :])
    ti_ref[0:1, :] = jnp.broadcast_to(t_val, (1, 128))
    ti_ref[1:2, :] = jnp.broadcast_to(l_eff, (1, 128))
    ti_ref[2:3, :] = bases
    ti_ref[3:8, :] = jnp.zeros((5, 128), jnp.int32)

    rel = rel_ref[...]
    h1 = jnp.maximum(jnp.dot(rel, w1_ref[...],
                             preferred_element_type=jnp.float32)
                     + b1_ref[...], 0.0)
    ctx = jnp.dot(h1, w2_ref[...],
                  preferred_element_type=jnp.float32) + b2_ref[...]
    cat = jnp.concatenate([ctx, rel], axis=1)
    a = jnp.maximum(jnp.dot(cat, wa1_ref[...],
                            preferred_element_type=jnp.float32)
                    + ba1_ref[...], 0.0)
    wgt = 1.0 / (1.0 + jnp.exp(-(jnp.dot(a, wa2_ref[...],
                                         preferred_element_type=jnp.float32)
                                 + ba2_ref[...])))
    fin = jnp.concatenate([ctx * wgt[0, 0], rel], axis=1)
    f1 = jnp.maximum(jnp.dot(fin, wc1_ref[...],
                             preferred_element_type=jnp.float32)
                     + bc1_ref[...], 0.0)
    fused = jnp.dot(f1, wc2_ref[...],
                    preferred_element_type=jnp.float32) + bc2_ref[...]
    c_out = 1.0 + jnp.tanh(jnp.mean(fused))
    cv_ref[...] = jnp.broadcast_to(c_out, (8, 128))


def _kc(u2d, rel_row, W1, b1, W2, b2, Wa1, ba1, Wa2, ba2, Wc1, bc1, Wc2, bc2):
    return pl.pallas_call(
        _kc_body,
        out_shape=[jax.ShapeDtypeStruct((8, 128), jnp.int32),
                   jax.ShapeDtypeStruct((8, 128), jnp.float32)],
    )(u2d, rel_row, W1, b1[None, :], W2, b2[None, :], Wa1, ba1[None, :],
      Wa2, ba2[None, :], Wc1, bc1[None, :], Wc2, bc2[None, :])


def _k4_body(src1d, dst1d, u1d, ti64, qe16, nmp, nm_sh,
             sidx, sidx2, didx, didx2, u_v, u_v2, ti_v, qe_v,
             sc1, sc2, sc3, run_v, zero_v, sem_i0, sem_i1):
    cid = lax.axis_index("c")
    sid = lax.axis_index("s")
    w = _worker(cid, sid)
    _zero_ref(zero_v, SLICE_N)
    pltpu.sync_copy(zero_v, nm_sh.at[pl.ds(sid * SLICE_N, SLICE_N)])
    pltpu.sync_copy(ti64, ti_v)
    pltpu.sync_copy(qe16, qe_v)
    t_b = ti_v[pl.ds(0, LANE)]
    l_b = ti_v[pl.ds(16, LANE)]
    tl_b = t_b + jnp.where(l_b > 0, 0, 1)
    brow = ti_v[pl.ds(32 + cid * LANE, LANE)]
    iv = lax.iota(jnp.int32, LANE)
    base = jnp.sum(jnp.where(iv == sid, brow, 0))
    run_v[...] = jnp.broadcast_to(base, (LANE,))
    qe_b = qe_v[...]
    plsc.subcore_barrier()

    start, nblk = _sched(w)
    sbuf = [sidx, sidx2]
    dbuf = [didx, didx2]
    ubuf = [u_v, u_v2]
    sem_i = [sem_i0, sem_i1]

    def fire(gi):
        es = (start + gi * GRP) * BLK
        p = gi % 2
        return [
            pltpu.async_copy(src1d.at[pl.ds(es, GBLK)], sbuf[p], sem_i[p]),
            pltpu.async_copy(dst1d.at[pl.ds(es, GBLK)], dbuf[p], sem_i[p]),
            pltpu.async_copy(u1d.at[pl.ds(es, GBLK)], ubuf[p], sem_i[p])]

    pend_in = fire(0)
    for gi in range(NGRP):
        p = gi % 2
        nxt = fire(gi + 1) if gi + 1 < NGRP else []
        for d in pend_in:
            d.wait()
        pend_in = nxt
        sidx_g = sbuf[p]
        didx_g = dbuf[p]
        u_g = ubuf[p]
        for bj in range(GRP):
            bi = gi * GRP + bj
            off = bj * BLK

            def sc(i, ah):
                s = pl.ds(off + i * LANE, LANE)
                sv = sidx_g[s]
                dv = didx_g[s]
                uu = u_g[s]
                h = (sv == qe_b) | (dv == qe_b) | (uu >= tl_b)
                return ah | h.astype(jnp.int32)

            ah = lax.fori_loop(0, BLK // LANE, sc,
                               jnp.zeros((LANE,), jnp.int32))
            nh = jnp.sum(ah)

            @pl.when((nh > 0) & (bi < nblk))
            def _():
                def vec(i, _):
                    s = pl.ds(off + i * LANE, LANE)
                    sv = sidx_g[s]
                    dv = didx_g[s]
                    uu = u_g[s]
                    eqm = (uu == t_b).astype(jnp.int32)
                    inc = plsc.cumsum(eqm)
                    run_b = run_v[...]
                    rank = run_b + inc - 1
                    contrib = (uu > t_b) | ((eqm != 0) & (rank < l_b))
                    v1 = (sv == qe_b).astype(jnp.int32)
                    v2 = (dv == qe_b).astype(jnp.int32)
                    vc = contrib.astype(jnp.int32)
                    n1 = jnp.sum(v1)
                    n2 = jnp.sum(v2)
                    n3 = jnp.sum(vc)

                    @pl.when(n1 > 0)
                    def _():
                        sc1[...] = v1
                        pltpu.sync_copy(sc1, nm_sh.at[dv], add=True)

                    @pl.when(n2 > 0)
                    def _():
                        sc2[...] = v2
                        pltpu.sync_copy(sc2, nm_sh.at[sv], add=True)

                    @pl.when(n3 > 0)
                    def _():
                        sc3[...] = vc
                        pltpu.sync_copy(sc3, nm_sh.at[sv], add=True)
                        pltpu.sync_copy(sc3, nm_sh.at[dv], add=True)

                    run_v[...] = run_b + jnp.broadcast_to(jnp.sum(eqm),
                                                          (LANE,))
                    return 0
                lax.fori_loop(0, BLK // LANE, vec, 0)
    plsc.subcore_barrier()
    pltpu.sync_copy(nm_sh.at[pl.ds(sid * SLICE_N, SLICE_N)],
                    nmp.at[cid, pl.ds(sid * SLICE_N, SLICE_N)])


def _k4(src1d, dst1d, u1d, ti64, qe16):
    kern = pl.kernel(
        _k4_body,
        out_type=jax.ShapeDtypeStruct((NC, N_PAD), jnp.int32),
        mesh=_mesh(),
        scratch_types=[
            pltpu.VMEM_SHARED((N_PAD,), jnp.int32),
            pltpu.VMEM((GBLK,), jnp.int32),
            pltpu.VMEM((GBLK,), jnp.int32),
            pltpu.VMEM((GBLK,), jnp.int32),
            pltpu.VMEM((GBLK,), jnp.int32),
            pltpu.VMEM((GBLK,), jnp.int32),
            pltpu.VMEM((GBLK,), jnp.int32),
            pltpu.VMEM((64,), jnp.int32),
            pltpu.VMEM((16,), jnp.int32),
            pltpu.VMEM((LANE,), jnp.int32),
            pltpu.VMEM((LANE,), jnp.int32),
            pltpu.VMEM((LANE,), jnp.int32),
            pltpu.VMEM((LANE,), jnp.int32),
            pltpu.VMEM((SLICE_N,), jnp.int32),
            pltpu.SemaphoreType.DMA,
            pltpu.SemaphoreType.DMA,
        ],
        **_SC_PARAMS,
    )
    return kern(src1d, dst1d, u1d, ti64, qe16)


def _k5_body(nmp, src1d, dst1d, qe16, cv16, out1d, nm_v, sA0, sA1, dA0, dA1,
             oo0, oo1, qe_v, cv_v, sem_i0, sem_i1, sem_o0, sem_o1):
    cid = lax.axis_index("c")
    sid = lax.axis_index("s")
    w = _worker(cid, sid)
    _merge_chunked(nmp, nm_v, sA1, sem_i0)
    pltpu.sync_copy(qe16, qe_v)
    pltpu.sync_copy(cv16, cv_v)
    qe_b = qe_v[...]
    c_b = cv_v[...]
    zero16 = jnp.zeros((LANE,), jnp.float32)

    start, nblk = _sched(w)
    sbuf = [sA0, sA1]
    dbuf = [dA0, dA1]
    obuf = [oo0, oo1]
    sem_i = [sem_i0, sem_i1]
    sem_o = [sem_o0, sem_o1]

    def fire(gi):
        es = (start + gi * GRP) * BLK
        p = gi % 2
        return [
            pltpu.async_copy(src1d.at[pl.ds(es, GBLK)], sbuf[p], sem_i[p]),
            pltpu.async_copy(dst1d.at[pl.ds(es, GBLK)], dbuf[p], sem_i[p])]

    pend_in = fire(0)
    pend_out = {0: [], 1: []}
    for gi in range(NGRP):
        p = gi % 2
        es = (start + gi * GRP) * BLK
        for d in pend_out[p]:
            d.wait()
        nxt = fire(gi + 1) if gi + 1 < NGRP else []
        for d in pend_in:
            d.wait()
        pend_in = nxt
        sA = sbuf[p]
        dA = dbuf[p]
        oo = obuf[p]

        def vec(i, _):
            s = pl.ds(i * LANE, LANE)
            sv = sA[s]
            dv = dA[s]
            ns16 = plsc.load_gather(nm_v, [sv])
            nd16 = plsc.load_gather(nm_v, [dv])
            msk = ((ns16 > 0) | (sv == qe_b)) & ((nd16 > 0) | (dv == qe_b))
            oo[s] = jnp.where(msk, c_b, zero16)
            return 0
        lax.fori_loop(0, GBLK // LANE, vec, 0)

        if gi < NGRP - 1:
            pend_out[p] = [pltpu.async_copy(oo, out1d.at[pl.ds(es, GBLK)],
                                            sem_o[p])]
        else:
            pend_out[p] = [pltpu.async_copy(
                oo.at[pl.ds(0, (GRP - 1) * BLK)],
                out1d.at[pl.ds(es, (GRP - 1) * BLK)], sem_o[p])]

            @pl.when(GRP * (NGRP - 1) + GRP - 1 < nblk)
            def _():
                pltpu.sync_copy(
                    oo.at[pl.ds((GRP - 1) * BLK, BLK)],
                    out1d.at[pl.ds(es + (GRP - 1) * BLK, BLK)])
    for p in (0, 1):
        for d in pend_out[p]:
            d.wait()


def _k5(nmp, src1d, dst1d, qe16, cv16):
    kern = pl.kernel(
        _k5_body,
        out_type=jax.ShapeDtypeStruct((E_ALLOC,), jnp.float32),
        mesh=_mesh(),
        scratch_types=[
            pltpu.VMEM((N_PAD,), jnp.int32),
            pltpu.VMEM((GBLK,), jnp.int32),
            pltpu.VMEM((GBLK,), jnp.int32),
            pltpu.VMEM((GBLK,), jnp.int32),
            pltpu.VMEM((GBLK,), jnp.int32),
            pltpu.VMEM((GBLK,), jnp.float32),
            pltpu.VMEM((GBLK,), jnp.float32),
            pltpu.VMEM((16,), jnp.int32),
            pltpu.VMEM((16,), jnp.float32),
            pltpu.SemaphoreType.DMA,
            pltpu.SemaphoreType.DMA,
            pltpu.SemaphoreType.DMA,
            pltpu.SemaphoreType.DMA,
        ],
        **_SC_PARAMS,
    )
    return kern(nmp, src1d, dst1d, qe16, cv16)


def kernel(edge_index, edge_type, query_relation, query_entity,
           relation_embeddings, W1, b1, W2, b2, Wa1, ba1, Wa2, ba2,
           Wc1, bc1, Wc2, bc2):
    src = edge_index[0].astype(jnp.int32)
    dst = edge_index[1].astype(jnp.int32)
    pad = E_ALLOC - E
    src_p = jnp.concatenate([src, jnp.zeros((pad,), jnp.int32)])
    dst_p = jnp.concatenate([dst, jnp.zeros((pad,), jnp.int32)])
    et_p = jnp.concatenate([edge_type.astype(jnp.int32),
                            jnp.full((pad,), -1, jnp.int32)])
    qr = jnp.asarray(query_relation, jnp.int32)
    qe = jnp.asarray(query_entity, jnp.int32)
    iq1d = (et_p == qr).astype(jnp.int32)
    senc = src_p | (iq1d * IQ_BIT)
    src2d = src_p.reshape(ROWS_ALLOC, 128)
    dst2d = dst_p.reshape(ROWS_ALLOC, 128)
    qe16 = jnp.full((16,), qe, jnp.int32)

    degp = _k1(src2d, dst2d)
    u1d = _k2(degp, senc, dst_p)

    rel_row = jnp.take(relation_embeddings, qr, axis=0)[None, :]
    ti, cv = _kc(u1d[:E_PAD].reshape(ROWS, 128), rel_row, W1, b1, W2, b2,
                 Wa1, ba1, Wa2, ba2, Wc1, bc1, Wc2, bc2)
    ti64 = jnp.concatenate([ti[0, :16], ti[1, :16], ti[2, :32]])
    cv16 = cv[0, :16]

    nmp = _k4(src_p, dst_p, u1d, ti64, qe16)
    out_p = _k5(nmp, src_p, dst_p, qe16, cv16)
    return out_p[:E]
